# baseline (speedup 1.0000x reference)
import functools

import numpy as np

import jax
import jax.numpy as jnp
from jax import lax
from jax.experimental import pallas as pl
from jax.experimental.pallas import tpu as pltpu
from jax.experimental.pallas import tpu_sc as plsc

B = 4096
L = 200
D = 16
V = 10000

NC, NS = 2, 16
NW = NC * NS
BPW = B // NW
NB = 16
NSB = BPW // NB
NHIST = 9
NSING = 15
NIT = NHIST * NSB
CH = 128
NCHUNK = NB * L // CH

_SINGLE_POS = [0, 2, 4, 6, 8, 10, 12, 13, 14, 16, 17, 18, 19, 20, 22]
_HIST_POS = [1, 3, 5, 7, 9, 11, 15, 21, 23]
_MY_ORDER = _SINGLE_POS + _HIST_POS
_ROW_PERM = np.concatenate([np.arange(p * D, (p + 1) * D) for p in _MY_ORDER])

_SINGLE_OFF = np.array([0, 1, 2, 3, 4, 5, 6, 6, 7, 8, 8, 9, 10, 11, 13],
                       dtype=np.int32) * V
_HIST_OFF = np.array([0, 1, 2, 3, 4, 5, 7, 12, 13], dtype=np.int32) * V


def _sc_body(table_hbm, hidx_hbm, sidx_hbm, out_hbm,
             hidx_v, rows_v, outv_v, sidx_v, srow_v, sem0, sem1, sem_s):
    wid = lax.axis_index("s") * NC + lax.axis_index("c")
    base_b = wid * BPW
    sems = (sem0, sem1)

    for s in range(NSING):
        pltpu.sync_copy(sidx_hbm.at[pl.ds(s * B + base_b, BPW)], sidx_v)
        pltpu.async_copy(table_hbm.at[sidx_v], srow_v, sem_s).wait()
        pltpu.sync_copy(srow_v,
                        out_hbm.at[pl.ds(base_b, BPW), pl.ds(s * D, D)])

    def fire(it, par):
        c = it // NSB
        sb = lax.rem(it, NSB)
        src = c * (B * L) + (base_b + sb * NB) * L
        pltpu.sync_copy(hidx_hbm.at[pl.ds(src, NB * L)], hidx_v.at[par])
        pltpu.async_copy(
            table_hbm.at[hidx_v.at[par]], rows_v.at[par], sems[par])

    def drain(par):
        pltpu.make_async_copy(
            table_hbm.at[pl.ds(0, NB * L)], rows_v.at[par], sems[par]).wait()

    def accum(it, par):
        c = it // NSB
        sb = lax.rem(it, NSB)

        def body(bl, carry):
            r0 = bl * L
            vals = [rows_v[par, r0 + u] for u in range(L)]
            while len(vals) > 1:
                nxt = [vals[i] + vals[i + 1]
                       for i in range(0, len(vals) - 1, 2)]
                if len(vals) % 2:
                    nxt.append(vals[-1])
                vals = nxt
            outv_v[bl] = vals[0] * (1.0 / L)
            return carry

        lax.fori_loop(0, NB, body, 0)
        pltpu.sync_copy(
            outv_v,
            out_hbm.at[pl.ds(base_b + sb * NB, NB),
                       pl.ds((NSING + c) * D, D)])

    fire(0, 0)

    def loop_body(j, carry):
        for par in range(2):
            it = j * 2 + par

            @pl.when(it + 1 < NIT)
            def _():
                fire(it + 1, 1 - par)

            drain(par)
            accum(it, par)
        return carry

    lax.fori_loop(0, NIT // 2, loop_body, 0)


@functools.cache
def _sc_embed():
    return pl.kernel(
        _sc_body,
        out_type=jax.ShapeDtypeStruct((B, 24 * D), jnp.float32),
        mesh=plsc.VectorSubcoreMesh(core_axis_name="c",
                                    subcore_axis_name="s"),
        compiler_params=pltpu.CompilerParams(use_tc_tiling_on_sc=False),
        scratch_types=[
            pltpu.VMEM((2, NB * L), jnp.int32),
            pltpu.VMEM((2, NB * L, D), jnp.float32),
            pltpu.VMEM((NB, D), jnp.float32),
            pltpu.VMEM((BPW,), jnp.int32),
            pltpu.VMEM((BPW, D), jnp.float32),
            pltpu.SemaphoreType.DMA,
            pltpu.SemaphoreType.DMA,
            pltpu.SemaphoreType.DMA,
        ],
    )


def _bn(x, g, b):
    mu = jnp.mean(x, axis=0, keepdims=True)
    var = jnp.mean((x - mu) ** 2, axis=0, keepdims=True)
    return (x - mu) / jnp.sqrt(var + 1e-5) * g + b


def _dense_body(emb_ref, dense_ref,
                w0a, w0b, b0, g0, be0, w1, b1, g1, be1, w2, b2, g2, be2,
                fw0a, fw0b, fw0c, fb0, fg0, fbe0, fw1, fb1, fg1, fbe1,
                fw2t, fb2, out_ref):
    emb = emb_ref[...]
    dense = dense_ref[...]

    x = (jnp.dot(emb, w0a[...], preferred_element_type=jnp.float32)
         + jnp.dot(dense, w0b[...], preferred_element_type=jnp.float32)
         + b0[...])
    x = _bn(jax.nn.relu(x), g0[...], be0[...])
    x = jnp.dot(x, w1[...], preferred_element_type=jnp.float32) + b1[...]
    x = _bn(jax.nn.relu(x), g1[...], be1[...])
    x = jnp.dot(x, w2[...], preferred_element_type=jnp.float32) + b2[...]
    dnn_out = _bn(jax.nn.relu(x), g2[...], be2[...])

    linear = jnp.sum(emb, axis=1, keepdims=True)
    s = emb[:, 0:D]
    ssq = s * s
    for c in range(1, 24):
        e = emb[:, c * D:(c + 1) * D]
        s = s + e
        ssq = ssq + e * e
    cross = 0.5 * (s * s - ssq)

    y = (jnp.dot(dnn_out, fw0a[...], preferred_element_type=jnp.float32)
         + linear * fw0b[...]
         + jnp.dot(cross, fw0c[...], preferred_element_type=jnp.float32)
         + fb0[...])
    y = _bn(jax.nn.relu(y), fg0[...], fbe0[...])
    y = jnp.dot(y, fw1[...], preferred_element_type=jnp.float32) + fb1[...]
    y = _bn(jax.nn.relu(y), fg1[...], fbe1[...])
    logit = jnp.sum(y * fw2t[...], axis=1, keepdims=True) + fb2[...]
    out_ref[...] = 1.0 / (1.0 + jnp.exp(-logit))


def kernel(shop_id, shop_id_list, item_id, item_id_list, category_1_id,
           category_1_id_list, merge_standard_food_id,
           merge_standard_food_id_list, brand_id, brand_id_list,
           shop_aoi_id, shop_aoi_id_list, shop_geohash_12, geohash12,
           shop_geohash_6, shop_geohash6_list, visit_city, city_id,
           user_id, district_id, times, timediff_list, time_type,
           time_type_list, table_share_0, table_share_1, table_share_2,
           table_share_3, table_share_4, table_share_5, table_share_6,
           table_share_7, table_share_8, table_user_id, table_district_id,
           table_times, table_timediff_list, table_type, dense,
           dnn_W0, dnn_b0, dnn_g0, dnn_beta0,
           dnn_W1, dnn_b1, dnn_g1, dnn_beta1,
           dnn_W2, dnn_b2, dnn_g2, dnn_beta2,
           fdnn_W0, fdnn_b0, fdnn_g0, fdnn_beta0,
           fdnn_W1, fdnn_b1, fdnn_g1, fdnn_beta1,
           fdnn_W2, fdnn_b2):
    tables = jnp.concatenate(
        [table_share_0, table_share_1, table_share_2, table_share_3,
         table_share_4, table_share_5, table_share_6, table_share_7,
         table_share_8, table_user_id, table_district_id, table_times,
         table_timediff_list, table_type], axis=0)

    hist = jnp.stack(
        [shop_id_list, item_id_list, category_1_id_list,
         merge_standard_food_id_list, brand_id_list, shop_aoi_id_list,
         shop_geohash6_list, timediff_list, time_type_list])
    hidx = (hist.astype(jnp.int32)
            + jnp.asarray(_HIST_OFF).reshape(NHIST, 1, 1)).reshape(-1)
    sing = jnp.stack(
        [shop_id, item_id, category_1_id, merge_standard_food_id, brand_id,
         shop_aoi_id, shop_geohash_12, geohash12, shop_geohash_6,
         visit_city, city_id, user_id, district_id, times, time_type])
    sidx = (sing.astype(jnp.int32)
            + jnp.asarray(_SINGLE_OFF).reshape(NSING, 1)).reshape(-1)

    emb = _sc_embed()(tables, hidx, sidx)

    w0p = dnn_W0[jnp.asarray(_ROW_PERM)]
    row = lambda v: v.reshape(1, -1)
    out = pl.pallas_call(
        _dense_body,
        out_shape=jax.ShapeDtypeStruct((B, 1), jnp.float32),
    )(emb, dense,
      w0p, dnn_W0[24 * D:], row(dnn_b0), row(dnn_g0), row(dnn_beta0),
      dnn_W1, row(dnn_b1), row(dnn_g1), row(dnn_beta1),
      dnn_W2, row(dnn_b2), row(dnn_g2), row(dnn_beta2),
      fdnn_W0[0:32], fdnn_W0[32:33], fdnn_W0[33:49],
      row(fdnn_b0), row(fdnn_g0), row(fdnn_beta0),
      fdnn_W1, row(fdnn_b1), row(fdnn_g1), row(fdnn_beta1),
      fdnn_W2.reshape(1, -1), row(fdnn_b2))
    return out.reshape(-1)


# speedup vs baseline: 4.8589x; 4.8589x over previous
import functools

import numpy as np

import jax
import jax.numpy as jnp
from jax import lax
from jax.experimental import pallas as pl
from jax.experimental.pallas import tpu as pltpu
from jax.experimental.pallas import tpu_sc as plsc

B = 4096
L = 200
D = 16
V = 10000

NC, NS = 2, 16
NW = NC * NS
NHIST = 9
NSING = 15
SHARDS = 4
BPS = B // SHARDS
BPW = B // NW
CB = 64
NJOB_H = BPS // CB
NJOB_T = BPW // CB
NJOB = NJOB_H + NJOB_T
TROWS = V * 8

_SINGLE_POS = [0, 2, 4, 6, 8, 10, 12, 13, 14, 16, 17, 18, 19, 20, 22]
_HIST_POS = [1, 3, 5, 7, 9, 11, 15, 21, 23]
_MY_ORDER = _SINGLE_POS + _HIST_POS
_ROW_PERM = np.concatenate([np.arange(p * D, (p + 1) * D) for p in _MY_ORDER])

_SINGLE_OFF = np.array([0, 1, 2, 3, 4, 5, 6, 6, 7, 8, 8, 9, 10, 11, 13],
                       dtype=np.int32) * V


def _gather_step(tbl_v, idx, accs):
    idx8 = idx << 3
    new = list(accs)
    for p in range(8):
        w = plsc.load_gather(tbl_v, [idx8 + p])
        bf = plsc.bitcast(w, jnp.bfloat16)
        e, o = plsc.unpack(bf, format=plsc.PackFormat.INTERLEAVED,
                           preferred_element_type=jnp.float32)
        new[2 * p] = new[2 * p] + e
        new[2 * p + 1] = new[2 * p + 1] + o
    return tuple(new)


def _accum_192(tbl_v, idx_ref, start, accs_init):
    def body(v, accs):
        return _gather_step(tbl_v, idx_ref[pl.ds(start + v * 16, 16)], accs)
    return lax.fori_loop(0, 12, body, accs_init)


def _accum_half(tbl_v, idx_ref, off, lo, accs):
    iota = lax.iota(jnp.int32, 16)
    keep = (iota < 8) if lo else (iota >= 8)
    idx = jnp.where(keep, idx_ref[pl.ds(off, 16)], 0)
    return _gather_step(tbl_v, idx, accs)


def _finalize(accs, stage_v):
    for dd in range(16):
        stage_v[pl.ds(dd * 16, 16)] = plsc.cumsum(accs[dd])
    lane15 = lax.iota(jnp.int32, 16) * 16 + 15
    return plsc.load_gather(stage_v, [lane15])


def _sc_body(table_hbm, tblp_hbm, hidx_hbm, sidx_hbm, out_hbm,
             tbl_v, cidx_v, outv_v, stage_v, sidx_v, srow_v, sem_s):
    wid = lax.axis_index("s") * NC + lax.axis_index("c")
    base_b = wid * BPW

    for s in range(NSING):
        pltpu.sync_copy(sidx_hbm.at[pl.ds(s * B + base_b, BPW)], sidx_v)
        pltpu.async_copy(table_hbm.at[sidx_v], srow_v, sem_s).wait()
        pltpu.sync_copy(srow_v,
                        out_hbm.at[pl.ds(base_b, BPW), pl.ds(s * D, D)])

    col = wid // SHARDS
    shard = lax.rem(wid, SHARDS)
    pltpu.sync_copy(tblp_hbm.at[pl.ds(col * TROWS, TROWS)],
                    tbl_v.at[pl.ds(0, TROWS)])
    pltpu.sync_copy(tblp_hbm.at[pl.ds(8 * TROWS, 48)],
                    tbl_v.at[pl.ds(TROWS, 48)])

    def job_body(k, carry):
        heavy = k < NJOB_H
        b0 = jnp.where(heavy, shard * BPS + k * CB,
                       base_b + (k - NJOB_H) * CB)
        src_col = jnp.where(heavy, col, 8)
        src = src_col * (B * L) + b0 * L
        pltpu.sync_copy(hidx_hbm.at[pl.ds(src, CB * L)], cidx_v)
        dstc = jnp.where(heavy, (NSING + col) * D, 23 * D)

        def pair_body(pr, carry2):
            base = pr * (2 * L)
            z16 = tuple(jnp.zeros((16,), jnp.float32) for _ in range(16))
            accs = _accum_192(tbl_v, cidx_v, base, z16)
            accs = _accum_half(tbl_v, cidx_v, base + 192, True, accs)
            outv_v[2 * pr] = _finalize(accs, stage_v) * (1.0 / L)
            accs = _accum_half(tbl_v, cidx_v, base + 192, False, z16)
            accs = _accum_192(tbl_v, cidx_v, base + 208, accs)
            outv_v[2 * pr + 1] = _finalize(accs, stage_v) * (1.0 / L)
            return carry2

        lax.fori_loop(0, CB // 2, pair_body, 0)
        pltpu.sync_copy(outv_v, out_hbm.at[pl.ds(b0, CB), pl.ds(dstc, D)])
        return carry

    lax.fori_loop(0, NJOB, job_body, 0)


@functools.cache
def _sc_embed():
    return pl.kernel(
        _sc_body,
        out_type=jax.ShapeDtypeStruct((B, 24 * D), jnp.float32),
        mesh=plsc.VectorSubcoreMesh(core_axis_name="c",
                                    subcore_axis_name="s"),
        compiler_params=pltpu.CompilerParams(use_tc_tiling_on_sc=False,
                                             needs_layout_passes=False),
        scratch_types=[
            pltpu.VMEM((TROWS + 48,), jnp.int32),
            pltpu.VMEM((CB * L,), jnp.int32),
            pltpu.VMEM((CB, D), jnp.float32),
            pltpu.VMEM((256,), jnp.float32),
            pltpu.VMEM((BPW,), jnp.int32),
            pltpu.VMEM((BPW, D), jnp.float32),
            pltpu.SemaphoreType.DMA,
        ],
    )


def _bn(x, g, b):
    mu = jnp.mean(x, axis=0, keepdims=True)
    var = jnp.mean((x - mu) ** 2, axis=0, keepdims=True)
    return (x - mu) / jnp.sqrt(var + 1e-5) * g + b


def _dense_body(emb_ref, dense_ref,
                w0a, w0b, b0, g0, be0, w1, b1, g1, be1, w2, b2, g2, be2,
                fw0a, fw0b, fw0c, fb0, fg0, fbe0, fw1, fb1, fg1, fbe1,
                fw2t, fb2, out_ref):
    emb = emb_ref[...]
    dense = dense_ref[...]

    x = (jnp.dot(emb, w0a[...], preferred_element_type=jnp.float32)
         + jnp.dot(dense, w0b[...], preferred_element_type=jnp.float32)
         + b0[...])
    x = _bn(jax.nn.relu(x), g0[...], be0[...])
    x = jnp.dot(x, w1[...], preferred_element_type=jnp.float32) + b1[...]
    x = _bn(jax.nn.relu(x), g1[...], be1[...])
    x = jnp.dot(x, w2[...], preferred_element_type=jnp.float32) + b2[...]
    dnn_out = _bn(jax.nn.relu(x), g2[...], be2[...])

    linear = jnp.sum(emb, axis=1, keepdims=True)
    s = emb[:, 0:D]
    ssq = s * s
    for c in range(1, 24):
        e = emb[:, c * D:(c + 1) * D]
        s = s + e
        ssq = ssq + e * e
    cross = 0.5 * (s * s - ssq)

    y = (jnp.dot(dnn_out, fw0a[...], preferred_element_type=jnp.float32)
         + linear * fw0b[...]
         + jnp.dot(cross, fw0c[...], preferred_element_type=jnp.float32)
         + fb0[...])
    y = _bn(jax.nn.relu(y), fg0[...], fbe0[...])
    y = jnp.dot(y, fw1[...], preferred_element_type=jnp.float32) + fb1[...]
    y = _bn(jax.nn.relu(y), fg1[...], fbe1[...])
    logit = jnp.sum(y * fw2t[...], axis=1, keepdims=True) + fb2[...]
    out_ref[...] = 1.0 / (1.0 + jnp.exp(-logit))


def kernel(shop_id, shop_id_list, item_id, item_id_list, category_1_id,
           category_1_id_list, merge_standard_food_id,
           merge_standard_food_id_list, brand_id, brand_id_list,
           shop_aoi_id, shop_aoi_id_list, shop_geohash_12, geohash12,
           shop_geohash_6, shop_geohash6_list, visit_city, city_id,
           user_id, district_id, times, timediff_list, time_type,
           time_type_list, table_share_0, table_share_1, table_share_2,
           table_share_3, table_share_4, table_share_5, table_share_6,
           table_share_7, table_share_8, table_user_id, table_district_id,
           table_times, table_timediff_list, table_type, dense,
           dnn_W0, dnn_b0, dnn_g0, dnn_beta0,
           dnn_W1, dnn_b1, dnn_g1, dnn_beta1,
           dnn_W2, dnn_b2, dnn_g2, dnn_beta2,
           fdnn_W0, fdnn_b0, fdnn_g0, fdnn_beta0,
           fdnn_W1, fdnn_b1, fdnn_g1, fdnn_beta1,
           fdnn_W2, fdnn_b2):
    tables = jnp.concatenate(
        [table_share_0, table_share_1, table_share_2, table_share_3,
         table_share_4, table_share_5, table_share_6, table_share_7,
         table_share_8, table_user_id, table_district_id, table_times,
         table_timediff_list, table_type], axis=0)

    heavy = jnp.concatenate(
        [table_share_0, table_share_1, table_share_2, table_share_3,
         table_share_4, table_share_5, table_share_7,
         table_timediff_list], axis=0)
    type_pad = jnp.concatenate(
        [table_type, jnp.zeros((2, D), jnp.float32)], axis=0)
    packed = jnp.concatenate([heavy, type_pad], axis=0)
    packed = lax.bitcast_convert_type(
        packed.astype(jnp.bfloat16).reshape(-1, 2), jnp.int32).reshape(-1)

    hist = jnp.stack(
        [shop_id_list, item_id_list, category_1_id_list,
         merge_standard_food_id_list, brand_id_list, shop_aoi_id_list,
         shop_geohash6_list, timediff_list,
         time_type_list + V])
    hidx = hist.astype(jnp.int32).reshape(-1)
    sing = jnp.stack(
        [shop_id, item_id, category_1_id, merge_standard_food_id, brand_id,
         shop_aoi_id, shop_geohash_12, geohash12, shop_geohash_6,
         visit_city, city_id, user_id, district_id, times, time_type])
    sidx = (sing.astype(jnp.int32)
            + jnp.asarray(_SINGLE_OFF).reshape(NSING, 1)).reshape(-1)

    emb = _sc_embed()(tables, packed, hidx, sidx)

    w0p = dnn_W0[jnp.asarray(_ROW_PERM)]
    row = lambda v: v.reshape(1, -1)
    out = pl.pallas_call(
        _dense_body,
        out_shape=jax.ShapeDtypeStruct((B, 1), jnp.float32),
    )(emb, dense,
      w0p, dnn_W0[24 * D:], row(dnn_b0), row(dnn_g0), row(dnn_beta0),
      dnn_W1, row(dnn_b1), row(dnn_g1), row(dnn_beta1),
      dnn_W2, row(dnn_b2), row(dnn_g2), row(dnn_beta2),
      fdnn_W0[0:32], fdnn_W0[32:33], fdnn_W0[33:49],
      row(fdnn_b0), row(fdnn_g0), row(fdnn_beta0),
      fdnn_W1, row(fdnn_b1), row(fdnn_g1), row(fdnn_beta1),
      fdnn_W2.reshape(1, -1), row(fdnn_b2))
    return out.reshape(-1)
